# Initial kernel scaffold; baseline (speedup 1.0000x reference)
import jax
import jax.numpy as jnp
from jax.experimental import pallas as pl


def kernel(x, edge_index, W1, b1, W2, b2, W3, b3, W4, b4, Wf, bf):
    raise NotImplementedError("write your pallas kernel here")


# speedup vs baseline: 29.3035x; 29.3035x over previous
import functools

import jax
import jax.numpy as jnp
from jax import lax
from jax.experimental import pallas as pl
from jax.experimental.pallas import tpu as pltpu
from jax.experimental.pallas import tpu_sc as plsc

N = 10000
E = 320000
D = 128
H = 16

TILES = 16
EPT = E // TILES
CH = 80
NCH = EPT // CH
ROWS_PT = 632
NPAD = TILES * ROWS_PT
DEG_PT = 640
NDEG = TILES * DEG_PT

_mesh = plsc.VectorSubcoreMesh(core_axis_name="c", subcore_axis_name="s",
                               num_cores=1)


@functools.partial(
    pl.kernel, mesh=_mesh,
    compiler_params=pltpu.CompilerParams(use_tc_tiling_on_sc=False),
    out_type=jax.ShapeDtypeStruct((NDEG,), jnp.float32),
    scratch_types=[
        pltpu.VMEM((NCH, CH), jnp.int32),
        pltpu.VMEM((CH,), jnp.float32),
        pltpu.VMEM((DEG_PT,), jnp.float32),
        pltpu.VMEM_SHARED((NDEG,), jnp.float32),
        pltpu.SemaphoreType.DMA,
    ],
)
def _sc_degree(dst_hbm, deg_hbm, didx, ones, stage, dtab, sem):
    wid = lax.axis_index("s")
    for j in range(CH // 16):
        ones[pl.ds(j * 16, 16)] = jnp.full((16,), 1.0, jnp.float32)
    for j in range(DEG_PT // 16):
        stage[pl.ds(j * 16, 16)] = jnp.zeros((16,), jnp.float32)
    pltpu.sync_copy(stage, dtab.at[pl.ds(wid * DEG_PT, DEG_PT)])
    pltpu.sync_copy(dst_hbm.at[wid], didx)
    plsc.subcore_barrier()

    def chunk(c, carry):
        pltpu.sync_copy(ones, dtab.at[didx.at[c]], add=True)
        return carry

    lax.fori_loop(0, NCH, chunk, 0)
    plsc.subcore_barrier()
    pltpu.sync_copy(dtab.at[pl.ds(wid * DEG_PT, DEG_PT)], stage)
    pltpu.sync_copy(stage, deg_hbm.at[pl.ds(wid * DEG_PT, DEG_PT)])


@functools.partial(
    pl.kernel, mesh=_mesh,
    compiler_params=pltpu.CompilerParams(use_tc_tiling_on_sc=False),
    out_type=jax.ShapeDtypeStruct((NPAD, H), jnp.float32),
    scratch_types=[
        pltpu.VMEM((NCH, CH), jnp.int32),
        pltpu.VMEM((NCH, CH), jnp.int32),
        pltpu.VMEM((CH, H), jnp.float32),
        pltpu.VMEM((ROWS_PT, H), jnp.float32),
        pltpu.VMEM((ROWS_PT, H), jnp.float32),
        pltpu.VMEM_SHARED((NPAD, H), jnp.float32),
        pltpu.VMEM_SHARED((NPAD, H), jnp.float32),
        pltpu.SemaphoreType.DMA,
    ],
)
def _sc_propagate(u_hbm, src_hbm, dst_hbm, s_hbm, sidx, didx, rows, stage,
                  ustage, stab, utab, sem):
    wid = lax.axis_index("s")

    def zrow(i, carry):
        stage[i, :] = jnp.zeros((16,), jnp.float32)
        return carry

    lax.fori_loop(0, ROWS_PT, zrow, 0)
    pltpu.sync_copy(stage, stab.at[pl.ds(wid * ROWS_PT, ROWS_PT)])
    pltpu.sync_copy(u_hbm.at[pl.ds(wid * ROWS_PT, ROWS_PT)], ustage)
    pltpu.sync_copy(ustage, utab.at[pl.ds(wid * ROWS_PT, ROWS_PT)])
    pltpu.sync_copy(src_hbm.at[wid], sidx)
    pltpu.sync_copy(dst_hbm.at[wid], didx)
    plsc.subcore_barrier()

    def chunk(c, carry):
        pltpu.async_copy(utab.at[sidx.at[c]], rows, sem).wait()
        pltpu.sync_copy(rows, stab.at[didx.at[c]], add=True)
        return carry

    lax.fori_loop(0, NCH, chunk, 0)
    plsc.subcore_barrier()
    pltpu.sync_copy(stab.at[pl.ds(wid * ROWS_PT, ROWS_PT)], stage)
    pltpu.sync_copy(stage, s_hbm.at[pl.ds(wid * ROWS_PT, ROWS_PT)])


def _tc_first_body(x_ref, w1_ref, deg_ref, u_ref, dinv_ref):
    dinv = lax.rsqrt(deg_ref[...] + 1.0)
    h = jnp.dot(x_ref[...], w1_ref[...], preferred_element_type=jnp.float32)
    u_ref[...] = h * dinv
    dinv_ref[...] = dinv


_tc_first = pl.pallas_call(
    _tc_first_body,
    out_shape=(jax.ShapeDtypeStruct((NPAD, H), jnp.float32),
               jax.ShapeDtypeStruct((NPAD, 1), jnp.float32)),
)


def _tc_mid_body(s_ref, u_ref, dinv_ref, b_ref, w_ref, un_ref):
    pre = dinv_ref[...] * (s_ref[...] + u_ref[...]) + b_ref[...]
    r = jnp.maximum(pre, 0.0)
    un_ref[...] = dinv_ref[...] * jnp.dot(
        r, w_ref[...], preferred_element_type=jnp.float32)


_tc_mid = pl.pallas_call(
    _tc_mid_body,
    out_shape=jax.ShapeDtypeStruct((NPAD, H), jnp.float32),
)


def _tc_final_body(s_ref, u_ref, dinv_ref, b_ref, wf_ref, bf_ref, out_ref):
    pre = dinv_ref[...] * (s_ref[...] + u_ref[...]) + b_ref[...]
    h4 = jnp.maximum(pre, 0.0)
    pooled = jnp.sum(h4[:N, :], axis=0, keepdims=True) * (1.0 / N)
    logits = jnp.dot(pooled, wf_ref[...],
                     preferred_element_type=jnp.float32) + bf_ref[...]
    m = jnp.max(logits, axis=1, keepdims=True)
    ex = jnp.exp(logits - m)
    out_ref[...] = (logits - m) - jnp.log(jnp.sum(ex, axis=1, keepdims=True))


_tc_final = pl.pallas_call(
    _tc_final_body,
    out_shape=jax.ShapeDtypeStruct((1, 16), jnp.float32),
)


def kernel(x, edge_index, W1, b1, W2, b2, W3, b3, W4, b4, Wf, bf):
    src3 = edge_index[0].astype(jnp.int32).reshape(TILES, NCH, CH)
    dst3 = edge_index[1].astype(jnp.int32).reshape(TILES, NCH, CH)

    deg_raw = _sc_degree(dst3)
    deg = jnp.pad(deg_raw[:N], (0, NPAD - N)).reshape(NPAD, 1)

    xpad = jnp.pad(x, ((0, NPAD - N), (0, 0)))
    u, dinv = _tc_first(xpad, W1, deg)

    s = _sc_propagate(u, src3, dst3)
    u = _tc_mid(s, u, dinv, b1.reshape(1, H), W2)
    s = _sc_propagate(u, src3, dst3)
    u = _tc_mid(s, u, dinv, b2.reshape(1, H), W3)
    s = _sc_propagate(u, src3, dst3)
    u = _tc_mid(s, u, dinv, b3.reshape(1, H), W4)
    s = _sc_propagate(u, src3, dst3)

    return _tc_final(s, u, dinv, b4.reshape(1, H), Wf, bf.reshape(1, 16))


# speedup vs baseline: 46.7926x; 1.5968x over previous
import functools

import jax
import jax.numpy as jnp
from jax import lax
from jax.experimental import pallas as pl
from jax.experimental.pallas import tpu as pltpu
from jax.experimental.pallas import tpu_sc as plsc

N = 10000
E = 320000
D = 128
H = 16

NC = 2
TILES = 32
CH = 128
NCH = 80
NBUF = 4
EPAD = TILES * NCH * CH
ROWS_PT = 320
NPAD = TILES * ROWS_PT
ROWS_PS = 640

_mesh = plsc.VectorSubcoreMesh(core_axis_name="c", subcore_axis_name="s",
                               num_cores=NC)


@functools.partial(
    pl.kernel, mesh=_mesh,
    compiler_params=pltpu.CompilerParams(use_tc_tiling_on_sc=False),
    out_type=jax.ShapeDtypeStruct((NC, NPAD), jnp.float32),
    scratch_types=[
        pltpu.VMEM((NCH, CH), jnp.int32),
        pltpu.VMEM((CH,), jnp.float32),
        pltpu.VMEM((ROWS_PS,), jnp.float32),
        pltpu.VMEM_SHARED((NPAD,), jnp.float32),
        pltpu.SemaphoreType.DMA,
    ],
)
def _sc_degree(dst_hbm, deg_hbm, didx, ones, stage, dtab, sem):
    cid = lax.axis_index("c")
    sid = lax.axis_index("s")
    wid = sid * NC + cid
    for j in range(CH // 16):
        ones[pl.ds(j * 16, 16)] = jnp.full((16,), 1.0, jnp.float32)
    for j in range(ROWS_PS // 16):
        stage[pl.ds(j * 16, 16)] = jnp.zeros((16,), jnp.float32)
    pltpu.sync_copy(stage, dtab.at[pl.ds(sid * ROWS_PS, ROWS_PS)])
    pltpu.sync_copy(dst_hbm.at[wid], didx)
    plsc.subcore_barrier()

    def chunk(c, carry):
        pltpu.sync_copy(ones, dtab.at[didx.at[c]], add=True)
        return carry

    lax.fori_loop(0, NCH, chunk, 0)
    plsc.subcore_barrier()
    pltpu.sync_copy(dtab.at[pl.ds(sid * ROWS_PS, ROWS_PS)], stage)
    pltpu.sync_copy(stage, deg_hbm.at[cid, pl.ds(sid * ROWS_PS, ROWS_PS)])


@functools.partial(
    pl.kernel, mesh=_mesh,
    compiler_params=pltpu.CompilerParams(use_tc_tiling_on_sc=False),
    out_type=jax.ShapeDtypeStruct((NC, NPAD, H), jnp.float32),
    scratch_types=[
        pltpu.VMEM((NCH, CH), jnp.int32),
        pltpu.VMEM((NCH, CH), jnp.int32),
        [pltpu.VMEM((CH, H), jnp.float32)] * NBUF,
        pltpu.VMEM((ROWS_PS, H), jnp.float32),
        pltpu.VMEM_SHARED((NPAD, H), jnp.float32),
        pltpu.VMEM_SHARED((NPAD, H), jnp.float32),
        pltpu.SemaphoreType.DMA,
        pltpu.SemaphoreType.DMA,
    ],
)
def _sc_propagate(u_hbm, src_hbm, dst_hbm, s_hbm, sidx, didx, rows, stage,
                  utab, stab, gsem, ssem):
    cid = lax.axis_index("c")
    sid = lax.axis_index("s")
    wid = sid * NC + cid

    def zrow(i, carry):
        stage[i, :] = jnp.zeros((16,), jnp.float32)
        return carry

    lax.fori_loop(0, ROWS_PS, zrow, 0)
    pltpu.sync_copy(stage, stab.at[pl.ds(sid * ROWS_PS, ROWS_PS)])
    pltpu.sync_copy(u_hbm.at[pl.ds(sid * ROWS_PS, ROWS_PS)], stage)
    pltpu.sync_copy(stage, utab.at[pl.ds(sid * ROWS_PS, ROWS_PS)])
    pltpu.sync_copy(src_hbm.at[wid], sidx)
    pltpu.sync_copy(dst_hbm.at[wid], didx)
    plsc.subcore_barrier()

    def group(g, carry):
        base = g * NBUF
        cps = [
            pltpu.async_copy(utab.at[sidx.at[base + k]], rows[k], gsem)
            for k in range(NBUF)
        ]
        for k in range(NBUF):
            cps[k].wait()
            pltpu.sync_copy(rows[k], stab.at[didx.at[base + k]], add=True)
        return carry

    lax.fori_loop(0, NCH // NBUF, group, 0)
    plsc.subcore_barrier()
    pltpu.sync_copy(stab.at[pl.ds(sid * ROWS_PS, ROWS_PS)], stage)
    pltpu.sync_copy(stage, s_hbm.at[cid, pl.ds(sid * ROWS_PS, ROWS_PS)])


def _tc_first_body(x_ref, w1_ref, deg_ref, u_ref, dinv_ref):
    deg = deg_ref[0] + deg_ref[1]
    dinv = lax.rsqrt(deg + 1.0)
    h = jnp.dot(x_ref[...], w1_ref[...], preferred_element_type=jnp.float32)
    u_ref[...] = h * dinv
    dinv_ref[...] = dinv


_tc_first = pl.pallas_call(
    _tc_first_body,
    out_shape=(jax.ShapeDtypeStruct((NPAD, H), jnp.float32),
               jax.ShapeDtypeStruct((NPAD, 1), jnp.float32)),
)


def _tc_mid_body(s_ref, u_ref, dinv_ref, b_ref, w_ref, un_ref):
    s = s_ref[0] + s_ref[1]
    pre = dinv_ref[...] * (s + u_ref[...]) + b_ref[...]
    r = jnp.maximum(pre, 0.0)
    un_ref[...] = dinv_ref[...] * jnp.dot(
        r, w_ref[...], preferred_element_type=jnp.float32)


_tc_mid = pl.pallas_call(
    _tc_mid_body,
    out_shape=jax.ShapeDtypeStruct((NPAD, H), jnp.float32),
)


def _tc_final_body(s_ref, u_ref, dinv_ref, b_ref, wf_ref, bf_ref, out_ref):
    s = s_ref[0] + s_ref[1]
    pre = dinv_ref[...] * (s + u_ref[...]) + b_ref[...]
    h4 = jnp.maximum(pre, 0.0)
    pooled = jnp.sum(h4[:N, :], axis=0, keepdims=True) * (1.0 / N)
    logits = jnp.dot(pooled, wf_ref[...],
                     preferred_element_type=jnp.float32) + bf_ref[...]
    m = jnp.max(logits, axis=1, keepdims=True)
    ex = jnp.exp(logits - m)
    out_ref[...] = (logits - m) - jnp.log(jnp.sum(ex, axis=1, keepdims=True))


_tc_final = pl.pallas_call(
    _tc_final_body,
    out_shape=jax.ShapeDtypeStruct((1, 16), jnp.float32),
)


def kernel(x, edge_index, W1, b1, W2, b2, W3, b3, W4, b4, Wf, bf):
    src = edge_index[0].astype(jnp.int32)
    dst = edge_index[1].astype(jnp.int32)
    npadlen = EPAD - E
    src3 = jnp.concatenate(
        [src, jnp.zeros((npadlen,), jnp.int32)]).reshape(TILES, NCH, CH)
    dst3 = jnp.concatenate(
        [dst, jnp.full((npadlen,), NPAD - 1, jnp.int32)]).reshape(
            TILES, NCH, CH)

    deg_raw = _sc_degree(dst3)
    deg2 = deg_raw.reshape(NC, NPAD, 1)

    xpad = jnp.pad(x, ((0, NPAD - N), (0, 0)))
    u, dinv = _tc_first(xpad, W1, deg2)

    s = _sc_propagate(u, src3, dst3)
    u = _tc_mid(s, u, dinv, b1.reshape(1, H), W2)
    s = _sc_propagate(u, src3, dst3)
    u = _tc_mid(s, u, dinv, b2.reshape(1, H), W3)
    s = _sc_propagate(u, src3, dst3)
    u = _tc_mid(s, u, dinv, b3.reshape(1, H), W4)
    s = _sc_propagate(u, src3, dst3)

    return _tc_final(s, u, dinv, b4.reshape(1, H), Wf, bf.reshape(1, 16))


# speedup vs baseline: 53.8172x; 1.1501x over previous
import functools

import jax
import jax.numpy as jnp
from jax import lax
from jax.experimental import pallas as pl
from jax.experimental.pallas import tpu as pltpu
from jax.experimental.pallas import tpu_sc as plsc

N = 10000
E = 320000
D = 128
H = 16

NC = 2
TILES = 32
CH = 80
NCH = E // TILES // CH
NBUF = 5
ROWS_PS = 640
NPAD = 16 * ROWS_PS
PR = NPAD // 8
PRN = N // 8

_mesh = plsc.VectorSubcoreMesh(core_axis_name="c", subcore_axis_name="s",
                               num_cores=NC)


@functools.partial(
    pl.kernel, mesh=_mesh,
    compiler_params=pltpu.CompilerParams(use_tc_tiling_on_sc=False),
    out_type=jax.ShapeDtypeStruct((NC, NPAD, H), jnp.float32),
    scratch_types=[
        pltpu.VMEM((NCH, CH), jnp.int32),
        pltpu.VMEM((CH,), jnp.float32),
        pltpu.VMEM((ROWS_PS,), jnp.float32),
        pltpu.VMEM((ROWS_PS, H), jnp.float32),
        pltpu.VMEM_SHARED((NPAD,), jnp.float32),
        pltpu.SemaphoreType.DMA,
    ],
)
def _sc_degree(dst_hbm, deg_hbm, didx, ones, stage, stage2, dtab, sem):
    cid = lax.axis_index("c")
    sid = lax.axis_index("s")
    wid = sid * NC + cid
    for j in range(CH // 16):
        ones[pl.ds(j * 16, 16)] = jnp.full((16,), 1.0, jnp.float32)
    for j in range(ROWS_PS // 16):
        stage[pl.ds(j * 16, 16)] = jnp.zeros((16,), jnp.float32)
    pltpu.sync_copy(stage, dtab.at[pl.ds(sid * ROWS_PS, ROWS_PS)])
    pltpu.sync_copy(dst_hbm.at[wid], didx)
    plsc.subcore_barrier()

    def chunk(c, carry):
        pltpu.sync_copy(ones, dtab.at[didx.at[c]], add=True)
        return carry

    lax.fori_loop(0, NCH, chunk, 0)
    plsc.subcore_barrier()
    pltpu.sync_copy(dtab.at[pl.ds(sid * ROWS_PS, ROWS_PS)], stage)

    def expand(j, carry):
        v = stage[pl.ds(j * 16, 16)]
        for l in range(16):
            stage2[j * 16 + l, :] = jnp.full((16,), v[l], jnp.float32)
        return carry

    lax.fori_loop(0, ROWS_PS // 16, expand, 0)
    pltpu.sync_copy(stage2, deg_hbm.at[cid, pl.ds(sid * ROWS_PS, ROWS_PS)])


@functools.partial(
    pl.kernel, mesh=_mesh,
    compiler_params=pltpu.CompilerParams(use_tc_tiling_on_sc=False),
    out_type=jax.ShapeDtypeStruct((NC, NPAD, H), jnp.float32),
    scratch_types=[
        pltpu.VMEM((NCH, CH), jnp.int32),
        pltpu.VMEM((NCH, CH), jnp.int32),
        [pltpu.VMEM((CH, H), jnp.float32)] * NBUF,
        pltpu.VMEM((ROWS_PS, H), jnp.float32),
        pltpu.VMEM_SHARED((NPAD, H), jnp.float32),
        pltpu.SemaphoreType.DMA,
    ],
)
def _sc_propagate(u_hbm, src_hbm, dst_hbm, s_hbm, sidx, didx, rows, stage,
                  stab, gsem):
    cid = lax.axis_index("c")
    sid = lax.axis_index("s")
    wid = sid * NC + cid

    def zrow(i, carry):
        stage[i, :] = jnp.zeros((16,), jnp.float32)
        return carry

    lax.fori_loop(0, ROWS_PS, zrow, 0)
    pltpu.sync_copy(stage, stab.at[pl.ds(sid * ROWS_PS, ROWS_PS)])
    pltpu.sync_copy(src_hbm.at[wid], sidx)
    pltpu.sync_copy(dst_hbm.at[wid], didx)
    plsc.subcore_barrier()

    def group(g, carry):
        base = g * NBUF
        cps = [
            pltpu.async_copy(u_hbm.at[sidx.at[base + k]], rows[k], gsem)
            for k in range(NBUF)
        ]
        for k in range(NBUF):
            cps[k].wait()
            pltpu.sync_copy(rows[k], stab.at[didx.at[base + k]], add=True)
        return carry

    lax.fori_loop(0, NCH // NBUF, group, 0)
    plsc.subcore_barrier()
    pltpu.sync_copy(stab.at[pl.ds(sid * ROWS_PS, ROWS_PS)], stage)
    pltpu.sync_copy(stage, s_hbm.at[cid, pl.ds(sid * ROWS_PS, ROWS_PS)])


def _tc_first_body(xr_ref, w1s_ref, deg_ref, u_ref, de_ref):
    dinvexp = lax.rsqrt(deg_ref[0] + deg_ref[1] + 1.0)
    hp = jnp.dot(xr_ref[...], w1s_ref[...],
                 preferred_element_type=jnp.float32)
    u_ref[0:PRN, :] = hp * dinvexp[0:PRN, :]
    u_ref[PRN:, :] = jnp.zeros((PR - PRN, 128), jnp.float32)
    de_ref[...] = dinvexp


_tc_first = pl.pallas_call(
    _tc_first_body,
    out_shape=(jax.ShapeDtypeStruct((PR, 128), jnp.float32),
               jax.ShapeDtypeStruct((PR, 128), jnp.float32)),
)


def _tc_mid_body(s_ref, u_ref, de_ref, bt_ref, wbd_ref, un_ref):
    de = de_ref[...]
    ss = s_ref[0] + s_ref[1] + u_ref[...]
    r = jnp.maximum(de * ss + bt_ref[...], 0.0)
    un_ref[...] = de * jnp.dot(r, wbd_ref[...],
                               preferred_element_type=jnp.float32)


_tc_mid = pl.pallas_call(
    _tc_mid_body,
    out_shape=jax.ShapeDtypeStruct((PR, 128), jnp.float32),
)


def _tc_final_body(s_ref, u_ref, de_ref, bt_ref, fold_ref, wf_ref, bf_ref,
                   out_ref):
    ss = s_ref[0] + s_ref[1] + u_ref[...]
    h4 = jnp.maximum(de_ref[...] * ss + bt_ref[...], 0.0)
    colsum = (jnp.sum(h4, axis=0, keepdims=True)
              - float(PR - PRN) * jnp.maximum(bt_ref[...], 0.0))
    pooled = jnp.dot(colsum, fold_ref[...],
                     preferred_element_type=jnp.float32) * (1.0 / N)
    logits = jnp.dot(pooled, wf_ref[...],
                     preferred_element_type=jnp.float32) + bf_ref[...]
    m = jnp.max(logits, axis=1, keepdims=True)
    ex = jnp.exp(logits - m)
    out_ref[...] = (logits - m) - jnp.log(jnp.sum(ex, axis=1, keepdims=True))


_tc_final = pl.pallas_call(
    _tc_final_body,
    out_shape=jax.ShapeDtypeStruct((1, 16), jnp.float32),
)


def kernel(x, edge_index, W1, b1, W2, b2, W3, b3, W4, b4, Wf, bf):
    src3 = edge_index[0].astype(jnp.int32).reshape(TILES, NCH, CH)
    dst3 = edge_index[1].astype(jnp.int32).reshape(TILES, NCH, CH)

    eye8 = jnp.eye(8, dtype=jnp.float32)
    w1s = jnp.kron(eye8, W1)
    fold = jnp.tile(jnp.eye(16, dtype=jnp.float32), (8, 1))

    degexp = _sc_degree(dst3).reshape(NC, PR, 128)
    u, dexp = _tc_first(x.reshape(PRN, 8 * D), w1s, degexp)

    for b, Wn in ((b1, W2), (b2, W3), (b3, W4)):
        s = _sc_propagate(u.reshape(NPAD, H), src3, dst3)
        u = _tc_mid(s.reshape(NC, PR, 128), u, dexp,
                    jnp.tile(b.reshape(1, H), (1, 8)), jnp.kron(eye8, Wn))
    s = _sc_propagate(u.reshape(NPAD, H), src3, dst3)

    return _tc_final(s.reshape(NC, PR, 128), u, dexp,
                     jnp.tile(b4.reshape(1, H), (1, 8)), fold, Wf,
                     bf.reshape(1, 16))


# speedup vs baseline: 70.3457x; 1.3071x over previous
import functools

import jax
import jax.numpy as jnp
from jax import lax
from jax.experimental import pallas as pl
from jax.experimental.pallas import tpu as pltpu
from jax.experimental.pallas import tpu_sc as plsc

N = 10000
E = 320000
D = 128
H = 16

NC = 2
TILES = 32
CH = 128
NCH = 80
EPAD = TILES * NCH * CH
NBUF = 4
ROWS_PS = 640
NPAD = 16 * ROWS_PS
PR = NPAD // 8
PRN = N // 8

_mesh = plsc.VectorSubcoreMesh(core_axis_name="c", subcore_axis_name="s",
                               num_cores=NC)


@functools.partial(
    pl.kernel, mesh=_mesh,
    compiler_params=pltpu.CompilerParams(use_tc_tiling_on_sc=False),
    out_type=jax.ShapeDtypeStruct((NC, NPAD, H), jnp.float32),
    scratch_types=[
        pltpu.VMEM((NCH, CH), jnp.int32),
        pltpu.VMEM((CH,), jnp.float32),
        pltpu.VMEM((ROWS_PS,), jnp.float32),
        pltpu.VMEM((ROWS_PS, H), jnp.float32),
        pltpu.VMEM_SHARED((NPAD,), jnp.float32),
        pltpu.SemaphoreType.DMA,
    ],
)
def _sc_degree(dst_hbm, deg_hbm, didx, ones, stage, stage2, dtab, sem):
    cid = lax.axis_index("c")
    sid = lax.axis_index("s")
    wid = sid * NC + cid
    for j in range(CH // 16):
        ones[pl.ds(j * 16, 16)] = jnp.full((16,), 1.0, jnp.float32)
    for j in range(ROWS_PS // 16):
        stage[pl.ds(j * 16, 16)] = jnp.zeros((16,), jnp.float32)
    pltpu.sync_copy(stage, dtab.at[pl.ds(sid * ROWS_PS, ROWS_PS)])
    pltpu.sync_copy(dst_hbm.at[wid], didx)
    plsc.subcore_barrier()

    def chunk(c, carry):
        pltpu.sync_copy(ones, dtab.at[didx.at[c]], add=True)
        return carry

    lax.fori_loop(0, NCH, chunk, 0)
    plsc.subcore_barrier()
    pltpu.sync_copy(dtab.at[pl.ds(sid * ROWS_PS, ROWS_PS)], stage)

    def expand(j, carry):
        v = stage[pl.ds(j * 16, 16)]
        for l in range(16):
            stage2[j * 16 + l, :] = jnp.full((16,), v[l], jnp.float32)
        return carry

    lax.fori_loop(0, ROWS_PS // 16, expand, 0)
    pltpu.sync_copy(stage2, deg_hbm.at[cid, pl.ds(sid * ROWS_PS, ROWS_PS)])


@functools.partial(
    pl.kernel, mesh=_mesh,
    compiler_params=pltpu.CompilerParams(use_tc_tiling_on_sc=False),
    out_type=jax.ShapeDtypeStruct((NC, NPAD, H), jnp.float32),
    scratch_types=[
        pltpu.VMEM((NCH, CH), jnp.int32),
        pltpu.VMEM((NCH, CH), jnp.int32),
        [pltpu.VMEM((CH, H), jnp.float32)] * NBUF,
        pltpu.VMEM((ROWS_PS, H), jnp.float32),
        pltpu.VMEM_SHARED((NPAD, H), jnp.float32),
        pltpu.VMEM_SHARED((NPAD, H), jnp.float32),
        pltpu.SemaphoreType.DMA,
        pltpu.SemaphoreType.DMA,
    ],
)
def _sc_propagate(u_hbm, src_hbm, dst_hbm, s_hbm, sidx, didx, rows, stage,
                  stab, utab, gsem, ssem):
    cid = lax.axis_index("c")
    sid = lax.axis_index("s")
    wid = sid * NC + cid

    def zrow(i, carry):
        stage[i, :] = jnp.zeros((16,), jnp.float32)
        return carry

    lax.fori_loop(0, ROWS_PS, zrow, 0)
    pltpu.sync_copy(stage, stab.at[pl.ds(sid * ROWS_PS, ROWS_PS)])
    pltpu.sync_copy(u_hbm.at[pl.ds(sid * ROWS_PS, ROWS_PS)], stage)
    pltpu.sync_copy(stage, utab.at[pl.ds(sid * ROWS_PS, ROWS_PS)])
    pltpu.sync_copy(src_hbm.at[wid], sidx)
    pltpu.sync_copy(dst_hbm.at[wid], didx)
    plsc.subcore_barrier()

    def group(g, carry):
        base = g * NBUF
        gs = [
            pltpu.async_copy(utab.at[sidx.at[base + k]], rows[k], gsem)
            for k in range(NBUF)
        ]
        scs = []
        for k in range(NBUF):
            gs[k].wait()
            scs.append(pltpu.async_copy(rows[k], stab.at[didx.at[base + k]],
                                        ssem, add=True))
        for cp in scs:
            cp.wait()
        return carry

    lax.fori_loop(0, NCH // NBUF, group, 0)
    plsc.subcore_barrier()
    pltpu.sync_copy(stab.at[pl.ds(sid * ROWS_PS, ROWS_PS)], stage)
    pltpu.sync_copy(stage, s_hbm.at[cid, pl.ds(sid * ROWS_PS, ROWS_PS)])


def _tc_first_body(xr_ref, w1s_ref, deg_ref, u_ref, de_ref):
    dinvexp = lax.rsqrt(deg_ref[0] + deg_ref[1] + 1.0)
    hp = jnp.dot(xr_ref[...], w1s_ref[...],
                 preferred_element_type=jnp.float32)
    u_ref[0:PRN, :] = hp * dinvexp[0:PRN, :]
    u_ref[PRN:, :] = jnp.zeros((PR - PRN, 128), jnp.float32)
    de_ref[...] = dinvexp


_tc_first = pl.pallas_call(
    _tc_first_body,
    out_shape=(jax.ShapeDtypeStruct((PR, 128), jnp.float32),
               jax.ShapeDtypeStruct((PR, 128), jnp.float32)),
)


def _tc_mid_body(s_ref, u_ref, de_ref, bt_ref, wbd_ref, un_ref):
    de = de_ref[...]
    ss = s_ref[0] + s_ref[1] + u_ref[...]
    r = jnp.maximum(de * ss + bt_ref[...], 0.0)
    un_ref[...] = de * jnp.dot(r, wbd_ref[...],
                               preferred_element_type=jnp.float32)


_tc_mid = pl.pallas_call(
    _tc_mid_body,
    out_shape=jax.ShapeDtypeStruct((PR, 128), jnp.float32),
)


def _tc_final_body(s_ref, u_ref, de_ref, bt_ref, fold_ref, wf_ref, bf_ref,
                   out_ref):
    ss = s_ref[0] + s_ref[1] + u_ref[...]
    h4 = jnp.maximum(de_ref[...] * ss + bt_ref[...], 0.0)
    colsum = (jnp.sum(h4, axis=0, keepdims=True)
              - float(PR - PRN) * jnp.maximum(bt_ref[...], 0.0))
    pooled = jnp.dot(colsum, fold_ref[...],
                     preferred_element_type=jnp.float32) * (1.0 / N)
    logits = jnp.dot(pooled, wf_ref[...],
                     preferred_element_type=jnp.float32) + bf_ref[...]
    m = jnp.max(logits, axis=1, keepdims=True)
    ex = jnp.exp(logits - m)
    out_ref[...] = (logits - m) - jnp.log(jnp.sum(ex, axis=1, keepdims=True))


_tc_final = pl.pallas_call(
    _tc_final_body,
    out_shape=jax.ShapeDtypeStruct((1, 16), jnp.float32),
)


def kernel(x, edge_index, W1, b1, W2, b2, W3, b3, W4, b4, Wf, bf):
    pe = EPAD - E
    src3 = jnp.concatenate(
        [edge_index[0].astype(jnp.int32), jnp.zeros((pe,), jnp.int32)]
    ).reshape(TILES, NCH, CH)
    dst3 = jnp.concatenate(
        [edge_index[1].astype(jnp.int32),
         jnp.full((pe,), NPAD - 1, jnp.int32)]
    ).reshape(TILES, NCH, CH)

    eye8 = jnp.eye(8, dtype=jnp.float32)
    w1s = jnp.kron(eye8, W1)
    fold = jnp.tile(jnp.eye(16, dtype=jnp.float32), (8, 1))

    degexp = _sc_degree(dst3).reshape(NC, PR, 128)
    u, dexp = _tc_first(x.reshape(PRN, 8 * D), w1s, degexp)

    for b, Wn in ((b1, W2), (b2, W3), (b3, W4)):
        s = _sc_propagate(u.reshape(NPAD, H), src3, dst3)
        u = _tc_mid(s.reshape(NC, PR, 128), u, dexp,
                    jnp.tile(b.reshape(1, H), (1, 8)), jnp.kron(eye8, Wn))
    s = _sc_propagate(u.reshape(NPAD, H), src3, dst3)

    return _tc_final(s.reshape(NC, PR, 128), u, dexp,
                     jnp.tile(b4.reshape(1, H), (1, 8)), fold, Wf,
                     bf.reshape(1, 16))


# speedup vs baseline: 72.6523x; 1.0328x over previous
import functools

import jax
import jax.numpy as jnp
from jax import lax
from jax.experimental import pallas as pl
from jax.experimental.pallas import tpu as pltpu
from jax.experimental.pallas import tpu_sc as plsc

N = 10000
E = 320000
D = 128
H = 16

NC = 2
TILES = 32
CH = 128
NCH = 80
EPAD = TILES * NCH * CH
NBUF = 8
ROWS_PS = 640
NPAD = 16 * ROWS_PS
PR = NPAD // 8
PRN = N // 8

_mesh = plsc.VectorSubcoreMesh(core_axis_name="c", subcore_axis_name="s",
                               num_cores=NC)


@functools.partial(
    pl.kernel, mesh=_mesh,
    compiler_params=pltpu.CompilerParams(use_tc_tiling_on_sc=False),
    out_type=jax.ShapeDtypeStruct((NC, NPAD, H), jnp.float32),
    scratch_types=[
        pltpu.VMEM((NCH, CH), jnp.int32),
        pltpu.VMEM((CH,), jnp.float32),
        pltpu.VMEM((ROWS_PS,), jnp.float32),
        pltpu.VMEM((ROWS_PS, H), jnp.float32),
        pltpu.VMEM_SHARED((NPAD,), jnp.float32),
        pltpu.SemaphoreType.DMA,
    ],
)
def _sc_degree(dst_hbm, deg_hbm, didx, ones, stage, stage2, dtab, sem):
    cid = lax.axis_index("c")
    sid = lax.axis_index("s")
    wid = sid * NC + cid
    for j in range(CH // 16):
        ones[pl.ds(j * 16, 16)] = jnp.full((16,), 1.0, jnp.float32)
    for j in range(ROWS_PS // 16):
        stage[pl.ds(j * 16, 16)] = jnp.zeros((16,), jnp.float32)
    pltpu.sync_copy(stage, dtab.at[pl.ds(sid * ROWS_PS, ROWS_PS)])
    pltpu.sync_copy(dst_hbm.at[wid], didx)
    plsc.subcore_barrier()

    def chunk(g, carry):
        base = g * 4
        cps = [
            pltpu.async_copy(ones, dtab.at[didx.at[base + k]], sem, add=True)
            for k in range(4)
        ]
        for cp in cps:
            cp.wait()
        return carry

    lax.fori_loop(0, NCH // 4, chunk, 0)
    plsc.subcore_barrier()
    pltpu.sync_copy(dtab.at[pl.ds(sid * ROWS_PS, ROWS_PS)], stage)

    def expand(j, carry):
        v = stage[pl.ds(j * 16, 16)]
        for l in range(16):
            stage2[j * 16 + l, :] = jnp.full((16,), v[l], jnp.float32)
        return carry

    lax.fori_loop(0, ROWS_PS // 16, expand, 0)
    pltpu.sync_copy(stage2, deg_hbm.at[cid, pl.ds(sid * ROWS_PS, ROWS_PS)])


@functools.partial(
    pl.kernel, mesh=_mesh,
    compiler_params=pltpu.CompilerParams(use_tc_tiling_on_sc=False),
    out_type=jax.ShapeDtypeStruct((NC, NPAD, H), jnp.float32),
    scratch_types=[
        pltpu.VMEM((NCH, CH), jnp.int32),
        pltpu.VMEM((NCH, CH), jnp.int32),
        [pltpu.VMEM((CH, H), jnp.float32)] * NBUF,
        pltpu.VMEM((ROWS_PS, H), jnp.float32),
        pltpu.VMEM_SHARED((NPAD, H), jnp.float32),
        pltpu.VMEM_SHARED((NPAD, H), jnp.float32),
        pltpu.SemaphoreType.DMA,
        pltpu.SemaphoreType.DMA,
    ],
)
def _sc_propagate(u_hbm, src_hbm, dst_hbm, s_hbm, sidx, didx, rows, stage,
                  stab, utab, gsem, ssem):
    cid = lax.axis_index("c")
    sid = lax.axis_index("s")
    wid = sid * NC + cid

    def zrow(i, carry):
        stage[i, :] = jnp.zeros((16,), jnp.float32)
        return carry

    lax.fori_loop(0, ROWS_PS, zrow, 0)
    pltpu.sync_copy(stage, stab.at[pl.ds(sid * ROWS_PS, ROWS_PS)])
    pltpu.sync_copy(u_hbm.at[pl.ds(sid * ROWS_PS, ROWS_PS)], stage)
    pltpu.sync_copy(stage, utab.at[pl.ds(sid * ROWS_PS, ROWS_PS)])
    pltpu.sync_copy(src_hbm.at[wid], sidx)
    pltpu.sync_copy(dst_hbm.at[wid], didx)
    plsc.subcore_barrier()

    def group(g, carry):
        base = g * NBUF
        gs = [
            pltpu.async_copy(utab.at[sidx.at[base + k]], rows[k], gsem)
            for k in range(NBUF)
        ]
        scs = []
        for k in range(NBUF):
            gs[k].wait()
            scs.append(pltpu.async_copy(rows[k], stab.at[didx.at[base + k]],
                                        ssem, add=True))
        for cp in scs:
            cp.wait()
        return carry

    lax.fori_loop(0, NCH // NBUF, group, 0)
    plsc.subcore_barrier()
    pltpu.sync_copy(stab.at[pl.ds(sid * ROWS_PS, ROWS_PS)], stage)
    pltpu.sync_copy(stage, s_hbm.at[cid, pl.ds(sid * ROWS_PS, ROWS_PS)])


def _tc_first_body(xr_ref, w1s_ref, deg_ref, u_ref, de_ref):
    dinvexp = lax.rsqrt(deg_ref[0] + deg_ref[1] + 1.0)
    hp = jnp.dot(xr_ref[...], w1s_ref[...],
                 preferred_element_type=jnp.float32)
    u_ref[0:PRN, :] = hp * dinvexp[0:PRN, :]
    u_ref[PRN:, :] = jnp.zeros((PR - PRN, 128), jnp.float32)
    de_ref[...] = dinvexp


_tc_first = pl.pallas_call(
    _tc_first_body,
    out_shape=(jax.ShapeDtypeStruct((PR, 128), jnp.float32),
               jax.ShapeDtypeStruct((PR, 128), jnp.float32)),
)


def _tc_mid_body(s_ref, u_ref, de_ref, bt_ref, wbd_ref, un_ref):
    de = de_ref[...]
    ss = s_ref[0] + s_ref[1] + u_ref[...]
    r = jnp.maximum(de * ss + bt_ref[...], 0.0)
    un_ref[...] = de * jnp.dot(r, wbd_ref[...],
                               preferred_element_type=jnp.float32)


_tc_mid = pl.pallas_call(
    _tc_mid_body,
    out_shape=jax.ShapeDtypeStruct((PR, 128), jnp.float32),
)


def _tc_final_body(s_ref, u_ref, de_ref, bt_ref, fold_ref, wf_ref, bf_ref,
                   out_ref):
    ss = s_ref[0] + s_ref[1] + u_ref[...]
    h4 = jnp.maximum(de_ref[...] * ss + bt_ref[...], 0.0)
    colsum = (jnp.sum(h4, axis=0, keepdims=True)
              - float(PR - PRN) * jnp.maximum(bt_ref[...], 0.0))
    pooled = jnp.dot(colsum, fold_ref[...],
                     preferred_element_type=jnp.float32) * (1.0 / N)
    logits = jnp.dot(pooled, wf_ref[...],
                     preferred_element_type=jnp.float32) + bf_ref[...]
    m = jnp.max(logits, axis=1, keepdims=True)
    ex = jnp.exp(logits - m)
    out_ref[...] = (logits - m) - jnp.log(jnp.sum(ex, axis=1, keepdims=True))


_tc_final = pl.pallas_call(
    _tc_final_body,
    out_shape=jax.ShapeDtypeStruct((1, 16), jnp.float32),
)


def kernel(x, edge_index, W1, b1, W2, b2, W3, b3, W4, b4, Wf, bf):
    pe = EPAD - E
    dst3 = jnp.concatenate(
        [edge_index[1].astype(jnp.int32),
         jnp.full((pe,), NPAD - 1, jnp.int32)]
    ).reshape(TILES, NCH, CH)
    degexp = _sc_degree(dst3).reshape(NC, PR, 128)
    src3 = jnp.concatenate(
        [edge_index[0].astype(jnp.int32), jnp.zeros((pe,), jnp.int32)]
    ).reshape(TILES, NCH, CH)

    eye8 = jnp.eye(8, dtype=jnp.float32)
    w1s = jnp.kron(eye8, W1)
    fold = jnp.tile(jnp.eye(16, dtype=jnp.float32), (8, 1))

    u, dexp = _tc_first(x.reshape(PRN, 8 * D), w1s, degexp)

    for b, Wn in ((b1, W2), (b2, W3), (b3, W4)):
        s = _sc_propagate(u.reshape(NPAD, H), src3, dst3)
        u = _tc_mid(s.reshape(NC, PR, 128), u, dexp,
                    jnp.tile(b.reshape(1, H), (1, 8)), jnp.kron(eye8, Wn))
    s = _sc_propagate(u.reshape(NPAD, H), src3, dst3)

    return _tc_final(s.reshape(NC, PR, 128), u, dexp,
                     jnp.tile(b4.reshape(1, H), (1, 8)), fold, Wf,
                     bf.reshape(1, 16))


# speedup vs baseline: 74.5278x; 1.0258x over previous
import functools

import jax
import jax.numpy as jnp
from jax import lax
from jax.experimental import pallas as pl
from jax.experimental.pallas import tpu as pltpu
from jax.experimental.pallas import tpu_sc as plsc

N = 10000
E = 320000
D = 128
H = 16

NC = 2
TILES = 32
CH = 128
NCH = 80
EPAD = TILES * NCH * CH
NBUF = 16
ROWS_PS = 640
NPAD = 16 * ROWS_PS
PR = NPAD // 8
PRN = N // 8

_mesh = plsc.VectorSubcoreMesh(core_axis_name="c", subcore_axis_name="s",
                               num_cores=NC)


@functools.partial(
    pl.kernel, mesh=_mesh,
    compiler_params=pltpu.CompilerParams(use_tc_tiling_on_sc=False),
    out_type=jax.ShapeDtypeStruct((NC, NPAD, H), jnp.float32),
    scratch_types=[
        pltpu.VMEM((NCH, CH), jnp.int32),
        pltpu.VMEM((CH,), jnp.float32),
        pltpu.VMEM((ROWS_PS,), jnp.float32),
        pltpu.VMEM((ROWS_PS, H), jnp.float32),
        pltpu.VMEM_SHARED((NPAD,), jnp.float32),
        pltpu.SemaphoreType.DMA,
    ],
)
def _sc_degree(dst_hbm, deg_hbm, didx, ones, stage, stage2, dtab, sem):
    cid = lax.axis_index("c")
    sid = lax.axis_index("s")
    wid = sid * NC + cid
    for j in range(CH // 16):
        ones[pl.ds(j * 16, 16)] = jnp.full((16,), 1.0, jnp.float32)
    for j in range(ROWS_PS // 16):
        stage[pl.ds(j * 16, 16)] = jnp.zeros((16,), jnp.float32)
    pltpu.sync_copy(stage, dtab.at[pl.ds(sid * ROWS_PS, ROWS_PS)])
    pltpu.sync_copy(dst_hbm.at[wid], didx)
    plsc.subcore_barrier()

    def chunk(g, carry):
        base = g * 4
        cps = [
            pltpu.async_copy(ones, dtab.at[didx.at[base + k]], sem, add=True)
            for k in range(4)
        ]
        for cp in cps:
            cp.wait()
        return carry

    lax.fori_loop(0, NCH // 4, chunk, 0)
    plsc.subcore_barrier()
    pltpu.sync_copy(dtab.at[pl.ds(sid * ROWS_PS, ROWS_PS)], stage)

    def expand(j, carry):
        v = stage[pl.ds(j * 16, 16)]
        for l in range(16):
            stage2[j * 16 + l, :] = jnp.full((16,), v[l], jnp.float32)
        return carry

    lax.fori_loop(0, ROWS_PS // 16, expand, 0)
    pltpu.sync_copy(stage2, deg_hbm.at[cid, pl.ds(sid * ROWS_PS, ROWS_PS)])


@functools.partial(
    pl.kernel, mesh=_mesh,
    compiler_params=pltpu.CompilerParams(use_tc_tiling_on_sc=False),
    out_type=jax.ShapeDtypeStruct((NC, NPAD, H), jnp.float32),
    scratch_types=[
        pltpu.VMEM((NCH, CH), jnp.int32),
        pltpu.VMEM((NCH, CH), jnp.int32),
        [pltpu.VMEM((CH, H), jnp.float32)] * NBUF,
        pltpu.VMEM((ROWS_PS, H), jnp.float32),
        pltpu.VMEM_SHARED((NPAD, H), jnp.float32),
        pltpu.VMEM_SHARED((NPAD, H), jnp.float32),
        pltpu.SemaphoreType.DMA,
        pltpu.SemaphoreType.DMA,
    ],
)
def _sc_propagate(u_hbm, src_hbm, dst_hbm, s_hbm, sidx, didx, rows, stage,
                  stab, utab, gsem, ssem):
    cid = lax.axis_index("c")
    sid = lax.axis_index("s")
    wid = sid * NC + cid

    def zrow(i, carry):
        stage[i, :] = jnp.zeros((16,), jnp.float32)
        return carry

    lax.fori_loop(0, ROWS_PS, zrow, 0)
    pltpu.sync_copy(stage, stab.at[pl.ds(sid * ROWS_PS, ROWS_PS)])
    pltpu.sync_copy(u_hbm.at[pl.ds(sid * ROWS_PS, ROWS_PS)], stage)
    pltpu.sync_copy(stage, utab.at[pl.ds(sid * ROWS_PS, ROWS_PS)])
    pltpu.sync_copy(src_hbm.at[wid], sidx)
    pltpu.sync_copy(dst_hbm.at[wid], didx)
    plsc.subcore_barrier()

    def group(g, carry):
        base = g * NBUF
        gs = [
            pltpu.async_copy(utab.at[sidx.at[base + k]], rows[k], gsem)
            for k in range(NBUF)
        ]
        scs = []
        for k in range(NBUF):
            gs[k].wait()
            scs.append(pltpu.async_copy(rows[k], stab.at[didx.at[base + k]],
                                        ssem, add=True))
        for cp in scs:
            cp.wait()
        return carry

    lax.fori_loop(0, NCH // NBUF, group, 0)
    plsc.subcore_barrier()
    pltpu.sync_copy(stab.at[pl.ds(sid * ROWS_PS, ROWS_PS)], stage)
    pltpu.sync_copy(stage, s_hbm.at[cid, pl.ds(sid * ROWS_PS, ROWS_PS)])


def _tc_first_body(xr_ref, w1s_ref, deg_ref, u_ref, de_ref):
    dinvexp = lax.rsqrt(deg_ref[0] + deg_ref[1] + 1.0)
    hp = jnp.dot(xr_ref[...], w1s_ref[...],
                 preferred_element_type=jnp.float32)
    u_ref[0:PRN, :] = hp * dinvexp[0:PRN, :]
    u_ref[PRN:, :] = jnp.zeros((PR - PRN, 128), jnp.float32)
    de_ref[...] = dinvexp


_tc_first = pl.pallas_call(
    _tc_first_body,
    out_shape=(jax.ShapeDtypeStruct((PR, 128), jnp.float32),
               jax.ShapeDtypeStruct((PR, 128), jnp.float32)),
)


def _tc_mid_body(s_ref, u_ref, de_ref, bt_ref, wbd_ref, un_ref):
    de = de_ref[...]
    ss = s_ref[0] + s_ref[1] + u_ref[...]
    r = jnp.maximum(de * ss + bt_ref[...], 0.0)
    un_ref[...] = de * jnp.dot(r, wbd_ref[...],
                               preferred_element_type=jnp.float32)


_tc_mid = pl.pallas_call(
    _tc_mid_body,
    out_shape=jax.ShapeDtypeStruct((PR, 128), jnp.float32),
)


def _tc_final_body(s_ref, u_ref, de_ref, bt_ref, fold_ref, wf_ref, bf_ref,
                   out_ref):
    ss = s_ref[0] + s_ref[1] + u_ref[...]
    h4 = jnp.maximum(de_ref[...] * ss + bt_ref[...], 0.0)
    colsum = (jnp.sum(h4, axis=0, keepdims=True)
              - float(PR - PRN) * jnp.maximum(bt_ref[...], 0.0))
    pooled = jnp.dot(colsum, fold_ref[...],
                     preferred_element_type=jnp.float32) * (1.0 / N)
    logits = jnp.dot(pooled, wf_ref[...],
                     preferred_element_type=jnp.float32) + bf_ref[...]
    m = jnp.max(logits, axis=1, keepdims=True)
    ex = jnp.exp(logits - m)
    out_ref[...] = (logits - m) - jnp.log(jnp.sum(ex, axis=1, keepdims=True))


_tc_final = pl.pallas_call(
    _tc_final_body,
    out_shape=jax.ShapeDtypeStruct((1, 16), jnp.float32),
)


def kernel(x, edge_index, W1, b1, W2, b2, W3, b3, W4, b4, Wf, bf):
    pe = EPAD - E
    dst3 = jnp.concatenate(
        [edge_index[1].astype(jnp.int32),
         jnp.full((pe,), NPAD - 1, jnp.int32)]
    ).reshape(TILES, NCH, CH)
    degexp = _sc_degree(dst3).reshape(NC, PR, 128)
    src3 = jnp.concatenate(
        [edge_index[0].astype(jnp.int32), jnp.zeros((pe,), jnp.int32)]
    ).reshape(TILES, NCH, CH)

    eye8 = jnp.eye(8, dtype=jnp.float32)
    w1s = jnp.kron(eye8, W1)
    fold = jnp.tile(jnp.eye(16, dtype=jnp.float32), (8, 1))

    u, dexp = _tc_first(x.reshape(PRN, 8 * D), w1s, degexp)

    for b, Wn in ((b1, W2), (b2, W3), (b3, W4)):
        s = _sc_propagate(u.reshape(NPAD, H), src3, dst3)
        u = _tc_mid(s.reshape(NC, PR, 128), u, dexp,
                    jnp.tile(b.reshape(1, H), (1, 8)), jnp.kron(eye8, Wn))
    s = _sc_propagate(u.reshape(NPAD, H), src3, dst3)

    return _tc_final(s.reshape(NC, PR, 128), u, dexp,
                     jnp.tile(b4.reshape(1, H), (1, 8)), fold, Wf,
                     bf.reshape(1, 16))


# speedup vs baseline: 82.5837x; 1.1081x over previous
import functools

import jax
import jax.numpy as jnp
from jax import lax
from jax.experimental import pallas as pl
from jax.experimental.pallas import tpu as pltpu
from jax.experimental.pallas import tpu_sc as plsc

N = 10000
E = 320000
D = 128
H = 16

NC = 2
TILES = 32
CH = 128
NCH = 80
EPAD = TILES * NCH * CH
NBUF = 16
ROWS_PS = 640
NPAD = 16 * ROWS_PS
PR = NPAD // 8
PRN = N // 8

_mesh = plsc.VectorSubcoreMesh(core_axis_name="c", subcore_axis_name="s",
                               num_cores=NC)


@functools.partial(
    pl.kernel, mesh=_mesh,
    compiler_params=pltpu.CompilerParams(use_tc_tiling_on_sc=False),
    out_type=jax.ShapeDtypeStruct((NC, NPAD, H), jnp.float32),
    scratch_types=[
        pltpu.VMEM((NCH, CH), jnp.int32),
        pltpu.VMEM((CH,), jnp.float32),
        pltpu.VMEM((ROWS_PS,), jnp.float32),
        pltpu.VMEM((ROWS_PS, H), jnp.float32),
        pltpu.VMEM_SHARED((NPAD,), jnp.float32),
        pltpu.SemaphoreType.DMA,
    ],
)
def _sc_degree(dst_hbm, deg_hbm, didx, ones, stage, stage2, dtab, sem):
    cid = lax.axis_index("c")
    sid = lax.axis_index("s")
    wid = sid * NC + cid
    for j in range(CH // 16):
        ones[pl.ds(j * 16, 16)] = jnp.full((16,), 1.0, jnp.float32)
    for j in range(ROWS_PS // 16):
        stage[pl.ds(j * 16, 16)] = jnp.zeros((16,), jnp.float32)
    pltpu.sync_copy(stage, dtab.at[pl.ds(sid * ROWS_PS, ROWS_PS)])
    pltpu.sync_copy(dst_hbm.at[wid], didx)
    plsc.subcore_barrier()

    def chunk(g, carry):
        base = g * 4
        cps = [
            pltpu.async_copy(ones, dtab.at[didx.at[base + k]], sem, add=True)
            for k in range(4)
        ]
        for cp in cps:
            cp.wait()
        return carry

    lax.fori_loop(0, NCH // 4, chunk, 0)
    plsc.subcore_barrier()
    pltpu.sync_copy(dtab.at[pl.ds(sid * ROWS_PS, ROWS_PS)], stage)

    def expand(j, carry):
        v = stage[pl.ds(j * 16, 16)]
        for l in range(16):
            stage2[j * 16 + l, :] = jnp.full((16,), v[l], jnp.float32)
        return carry

    lax.fori_loop(0, ROWS_PS // 16, expand, 0)
    pltpu.sync_copy(stage2, deg_hbm.at[cid, pl.ds(sid * ROWS_PS, ROWS_PS)])


@functools.partial(
    pl.kernel, mesh=_mesh,
    compiler_params=pltpu.CompilerParams(use_tc_tiling_on_sc=False),
    out_type=jax.ShapeDtypeStruct((NC, NPAD, H), jnp.float32),
    scratch_types=[
        pltpu.VMEM((NCH, CH), jnp.int32),
        pltpu.VMEM((NCH, CH), jnp.int32),
        [pltpu.VMEM((CH, H), jnp.float32)] * NBUF,
        pltpu.VMEM((ROWS_PS, H), jnp.float32),
        pltpu.VMEM_SHARED((NPAD, H), jnp.float32),
        pltpu.VMEM_SHARED((NPAD, H), jnp.float32),
        pltpu.SemaphoreType.DMA,
        pltpu.SemaphoreType.DMA,
    ],
)
def _sc_propagate(u_hbm, src_hbm, dst_hbm, s_hbm, sidx, didx, rows, stage,
                  stab, utab, gsem, ssem):
    cid = lax.axis_index("c")
    sid = lax.axis_index("s")
    wid = sid * NC + cid

    cp_s = pltpu.async_copy(src_hbm.at[wid], sidx, gsem)
    cp_d = pltpu.async_copy(dst_hbm.at[wid], didx, gsem)
    cp_u = pltpu.async_copy(u_hbm.at[pl.ds(sid * ROWS_PS, ROWS_PS)], stage,
                            ssem)

    def zrow(i, carry):
        rows[0][i % CH, :] = jnp.zeros((16,), jnp.float32)
        return carry

    lax.fori_loop(0, CH, zrow, 0)
    cp_u.wait()
    pltpu.sync_copy(stage, utab.at[pl.ds(sid * ROWS_PS, ROWS_PS)])
    for q in range(ROWS_PS // CH):
        pltpu.sync_copy(rows[0], stab.at[pl.ds(sid * ROWS_PS + q * CH, CH)])
    cp_s.wait()
    cp_d.wait()
    plsc.subcore_barrier()

    def group(g, carry):
        base = g * NBUF
        gs = [
            pltpu.async_copy(utab.at[sidx.at[base + k]], rows[k], gsem)
            for k in range(NBUF)
        ]
        scs = []
        for k in range(NBUF):
            gs[k].wait()
            scs.append(pltpu.async_copy(rows[k], stab.at[didx.at[base + k]],
                                        ssem, add=True))
        for cp in scs:
            cp.wait()
        return carry

    lax.fori_loop(0, NCH // NBUF, group, 0)
    plsc.subcore_barrier()
    pltpu.sync_copy(stab.at[pl.ds(sid * ROWS_PS, ROWS_PS)], stage)
    pltpu.sync_copy(stage, s_hbm.at[cid, pl.ds(sid * ROWS_PS, ROWS_PS)])


def _tc_first_body(xr_ref, w1s_ref, deg_ref, u_ref, de_ref):
    dinvexp = lax.rsqrt(deg_ref[0] + deg_ref[1] + 1.0)
    hp = jnp.dot(xr_ref[...], w1s_ref[...],
                 preferred_element_type=jnp.float32)
    u_ref[0:PRN, :] = hp * dinvexp[0:PRN, :]
    u_ref[PRN:, :] = jnp.zeros((PR - PRN, 128), jnp.float32)
    de_ref[...] = dinvexp


_tc_first = pl.pallas_call(
    _tc_first_body,
    out_shape=(jax.ShapeDtypeStruct((PR, 128), jnp.float32),
               jax.ShapeDtypeStruct((PR, 128), jnp.float32)),
)


def _tc_mid_body(s_ref, u_ref, de_ref, bt_ref, wbd_ref, un_ref):
    de = de_ref[...]
    ss = s_ref[0] + s_ref[1] + u_ref[...]
    r = jnp.maximum(de * ss + bt_ref[...], 0.0)
    un_ref[...] = de * jnp.dot(r, wbd_ref[...],
                               preferred_element_type=jnp.float32)


_tc_mid = pl.pallas_call(
    _tc_mid_body,
    out_shape=jax.ShapeDtypeStruct((PR, 128), jnp.float32),
)


def _tc_final_body(s_ref, u_ref, de_ref, bt_ref, fold_ref, wf_ref, bf_ref,
                   out_ref):
    ss = s_ref[0] + s_ref[1] + u_ref[...]
    h4 = jnp.maximum(de_ref[...] * ss + bt_ref[...], 0.0)
    colsum = (jnp.sum(h4, axis=0, keepdims=True)
              - float(PR - PRN) * jnp.maximum(bt_ref[...], 0.0))
    pooled = jnp.dot(colsum, fold_ref[...],
                     preferred_element_type=jnp.float32) * (1.0 / N)
    logits = jnp.dot(pooled, wf_ref[...],
                     preferred_element_type=jnp.float32) + bf_ref[...]
    m = jnp.max(logits, axis=1, keepdims=True)
    ex = jnp.exp(logits - m)
    out_ref[...] = (logits - m) - jnp.log(jnp.sum(ex, axis=1, keepdims=True))


_tc_final = pl.pallas_call(
    _tc_final_body,
    out_shape=jax.ShapeDtypeStruct((1, 16), jnp.float32),
)


def kernel(x, edge_index, W1, b1, W2, b2, W3, b3, W4, b4, Wf, bf):
    pe = EPAD - E
    dst3 = jnp.concatenate(
        [edge_index[1].astype(jnp.int32),
         jnp.full((pe,), NPAD - 1, jnp.int32)]
    ).reshape(TILES, NCH, CH)
    degexp = _sc_degree(dst3).reshape(NC, PR, 128)
    src3 = jnp.concatenate(
        [edge_index[0].astype(jnp.int32), jnp.zeros((pe,), jnp.int32)]
    ).reshape(TILES, NCH, CH)

    eye8 = jnp.eye(8, dtype=jnp.float32)
    w1s = jnp.kron(eye8, W1)
    fold = jnp.tile(jnp.eye(16, dtype=jnp.float32), (8, 1))

    u, dexp = _tc_first(x.reshape(PRN, 8 * D), w1s, degexp)

    for b, Wn in ((b1, W2), (b2, W3), (b3, W4)):
        s = _sc_propagate(u.reshape(NPAD, H), src3, dst3)
        u = _tc_mid(s.reshape(NC, PR, 128), u, dexp,
                    jnp.tile(b.reshape(1, H), (1, 8)), jnp.kron(eye8, Wn))
    s = _sc_propagate(u.reshape(NPAD, H), src3, dst3)

    return _tc_final(s.reshape(NC, PR, 128), u, dexp,
                     jnp.tile(b4.reshape(1, H), (1, 8)), fold, Wf,
                     bf.reshape(1, 16))


# speedup vs baseline: 90.5792x; 1.0968x over previous
import functools

import jax
import jax.numpy as jnp
from jax import lax
from jax.experimental import pallas as pl
from jax.experimental.pallas import tpu as pltpu
from jax.experimental.pallas import tpu_sc as plsc

N = 10000
E = 320000
D = 128
H = 16

NC = 2
TILES = 32
CH = 128
NCH = 80
EPAD = TILES * NCH * CH
NBUF = 16
ROWS_PS = 640
NPAD = 16 * ROWS_PS
PR = NPAD // 8
PRN = N // 8

_mesh = plsc.VectorSubcoreMesh(core_axis_name="c", subcore_axis_name="s",
                               num_cores=NC)


@functools.partial(
    pl.kernel, mesh=_mesh,
    compiler_params=pltpu.CompilerParams(use_tc_tiling_on_sc=False),
    out_type=jax.ShapeDtypeStruct((NC, NPAD, H), jnp.float32),
    scratch_types=[
        pltpu.VMEM((NCH, CH), jnp.int32),
        pltpu.VMEM((CH,), jnp.float32),
        pltpu.VMEM((ROWS_PS,), jnp.float32),
        pltpu.VMEM((ROWS_PS, H), jnp.float32),
        pltpu.VMEM_SHARED((NPAD,), jnp.float32),
        pltpu.SemaphoreType.DMA,
    ],
)
def _sc_degree(dst_hbm, deg_hbm, didx, ones, stage, stage2, dtab, sem):
    cid = lax.axis_index("c")
    sid = lax.axis_index("s")
    wid = sid * NC + cid
    for j in range(CH // 16):
        ones[pl.ds(j * 16, 16)] = jnp.full((16,), 1.0, jnp.float32)
    for j in range(ROWS_PS // 16):
        stage[pl.ds(j * 16, 16)] = jnp.zeros((16,), jnp.float32)
    pltpu.sync_copy(stage, dtab.at[pl.ds(sid * ROWS_PS, ROWS_PS)])
    pltpu.sync_copy(dst_hbm.at[wid], didx)
    plsc.subcore_barrier()

    def chunk(g, carry):
        base = g * 4
        cps = [
            pltpu.async_copy(ones, dtab.at[didx.at[base + k]], sem, add=True)
            for k in range(4)
        ]
        for cp in cps:
            cp.wait()
        return carry

    lax.fori_loop(0, NCH // 4, chunk, 0)
    plsc.subcore_barrier()
    pltpu.sync_copy(dtab.at[pl.ds(sid * ROWS_PS, ROWS_PS)], stage)

    def expand(j, carry):
        v = stage[pl.ds(j * 16, 16)]
        for l in range(16):
            stage2[j * 16 + l, :] = jnp.full((16,), v[l], jnp.float32)
        return carry

    lax.fori_loop(0, ROWS_PS // 16, expand, 0)
    pltpu.sync_copy(stage2, deg_hbm.at[cid, pl.ds(sid * ROWS_PS, ROWS_PS)])


@functools.partial(
    pl.kernel, mesh=_mesh,
    compiler_params=pltpu.CompilerParams(use_tc_tiling_on_sc=False),
    out_type=jax.ShapeDtypeStruct((NC, NPAD, H), jnp.float32),
    scratch_types=[
        pltpu.VMEM((NCH, CH), jnp.int32),
        pltpu.VMEM((NCH, CH), jnp.int32),
        [pltpu.VMEM((CH, H), jnp.float32)] * NBUF,
        pltpu.VMEM((ROWS_PS, H), jnp.float32),
        pltpu.VMEM_SHARED((NPAD, H), jnp.float32),
        pltpu.VMEM_SHARED((NPAD, H), jnp.float32),
        pltpu.SemaphoreType.DMA,
        pltpu.SemaphoreType.DMA,
    ],
)
def _sc_propagate(u_hbm, src_hbm, dst_hbm, s_hbm, sidx, didx, rows, stage,
                  stab, utab, gsem, ssem):
    cid = lax.axis_index("c")
    sid = lax.axis_index("s")
    wid = sid * NC + cid

    cp_s = pltpu.async_copy(src_hbm.at[wid], sidx, gsem)
    cp_d = pltpu.async_copy(dst_hbm.at[wid], didx, gsem)
    cp_u = pltpu.async_copy(u_hbm.at[pl.ds(sid * ROWS_PS, ROWS_PS)], stage,
                            ssem)

    def zrow(i, carry):
        rows[0][i % CH, :] = jnp.zeros((16,), jnp.float32)
        return carry

    lax.fori_loop(0, CH, zrow, 0)
    cp_u.wait()
    pltpu.sync_copy(stage, utab.at[pl.ds(sid * ROWS_PS, ROWS_PS)])
    for q in range(ROWS_PS // CH):
        pltpu.sync_copy(rows[0], stab.at[pl.ds(sid * ROWS_PS + q * CH, CH)])
    cp_s.wait()
    cp_d.wait()
    plsc.subcore_barrier()

    def group(g, carry):
        base = g * NBUF
        gs = [
            pltpu.async_copy(utab.at[sidx.at[base + k]], rows[k], gsem)
            for k in range(NBUF)
        ]
        scs = []
        for k in range(NBUF):
            gs[k].wait()
            scs.append(pltpu.async_copy(rows[k], stab.at[didx.at[base + k]],
                                        ssem, add=True))
        for cp in scs:
            cp.wait()
        return carry

    lax.fori_loop(0, NCH // NBUF, group, 0)
    plsc.subcore_barrier()
    pltpu.sync_copy(stab.at[pl.ds(sid * ROWS_PS, ROWS_PS)], stage)
    pltpu.sync_copy(stage, s_hbm.at[cid, pl.ds(sid * ROWS_PS, ROWS_PS)])


def _tc_first_body(xr_ref, w1s_ref, deg_ref, u_ref, de_ref):
    dinvexp = lax.rsqrt(deg_ref[0] + deg_ref[1] + 1.0)
    hp = jnp.dot(xr_ref[...], w1s_ref[...],
                 preferred_element_type=jnp.float32)
    u_ref[0:PRN, :] = hp * dinvexp[0:PRN, :]
    u_ref[PRN:, :] = jnp.zeros((PR - PRN, 128), jnp.float32)
    de_ref[...] = dinvexp


_tc_first = pl.pallas_call(
    _tc_first_body,
    out_shape=(jax.ShapeDtypeStruct((PR, 128), jnp.float32),
               jax.ShapeDtypeStruct((PR, 128), jnp.float32)),
)


def _tc_mid_body(s_ref, u_ref, de_ref, bt_ref, wbd_ref, un_ref):
    de = de_ref[...]
    ss = s_ref[0] + s_ref[1] + u_ref[...]
    r = jnp.maximum(de * ss + bt_ref[...], 0.0)
    un_ref[...] = de * jnp.dot(r, wbd_ref[...],
                               preferred_element_type=jnp.float32)


_tc_mid = pl.pallas_call(
    _tc_mid_body,
    out_shape=jax.ShapeDtypeStruct((PR, 128), jnp.float32),
)


def _tc_final_body(s_ref, u_ref, de_ref, bt_ref, fold_ref, wf_ref, bf_ref,
                   out_ref):
    ss = s_ref[0] + s_ref[1] + u_ref[...]
    h4 = jnp.maximum(de_ref[...] * ss + bt_ref[...], 0.0)
    colsum = (jnp.sum(h4, axis=0, keepdims=True)
              - float(PR - PRN) * jnp.maximum(bt_ref[...], 0.0))
    pooled = jnp.dot(colsum, fold_ref[...],
                     preferred_element_type=jnp.float32) * (1.0 / N)
    logits = jnp.dot(pooled, wf_ref[...],
                     preferred_element_type=jnp.float32) + bf_ref[...]
    m = jnp.max(logits, axis=1, keepdims=True)
    ex = jnp.exp(logits - m)
    out_ref[...] = (logits - m) - jnp.log(jnp.sum(ex, axis=1, keepdims=True))


_tc_final = pl.pallas_call(
    _tc_final_body,
    out_shape=jax.ShapeDtypeStruct((1, 16), jnp.float32),
)


def kernel(x, edge_index, W1, b1, W2, b2, W3, b3, W4, b4, Wf, bf):
    ept = EPAD // TILES - E // TILES
    srcr = edge_index[0].astype(jnp.int32).reshape(TILES, E // TILES)
    dstr = edge_index[1].astype(jnp.int32).reshape(TILES, E // TILES)
    dums = jnp.zeros((TILES, ept), jnp.int32)
    dumd = jnp.tile(jnp.arange(N, N + ept, dtype=jnp.int32)[None, :],
                    (TILES, 1))
    dst3 = jnp.concatenate([dstr, dumd], axis=1).reshape(TILES, NCH, CH)
    degexp = _sc_degree(dst3).reshape(NC, PR, 128)
    src3 = jnp.concatenate([srcr, dums], axis=1).reshape(TILES, NCH, CH)

    eye8 = jnp.eye(8, dtype=jnp.float32)
    w1s = jnp.kron(eye8, W1)
    fold = jnp.tile(jnp.eye(16, dtype=jnp.float32), (8, 1))

    u, dexp = _tc_first(x.reshape(PRN, 8 * D), w1s, degexp)

    for b, Wn in ((b1, W2), (b2, W3), (b3, W4)):
        s = _sc_propagate(u.reshape(NPAD, H), src3, dst3)
        u = _tc_mid(s.reshape(NC, PR, 128), u, dexp,
                    jnp.tile(b.reshape(1, H), (1, 8)), jnp.kron(eye8, Wn))
    s = _sc_propagate(u.reshape(NPAD, H), src3, dst3)

    return _tc_final(s.reshape(NC, PR, 128), u, dexp,
                     jnp.tile(b4.reshape(1, H), (1, 8)), fold, Wf,
                     bf.reshape(1, 16))


# speedup vs baseline: 93.4004x; 1.0311x over previous
import functools

import jax
import jax.numpy as jnp
from jax import lax
from jax.experimental import pallas as pl
from jax.experimental.pallas import tpu as pltpu
from jax.experimental.pallas import tpu_sc as plsc

N = 10000
E = 320000
D = 128
H = 16

NC = 2
TILES = 32
CH = 128
NCH = 80
EPAD = TILES * NCH * CH
NBUF = 16
ROWS_PS = 640
NPAD = 16 * ROWS_PS
PR = NPAD // 8
PRN = N // 8

_mesh = plsc.VectorSubcoreMesh(core_axis_name="c", subcore_axis_name="s",
                               num_cores=NC)


@functools.partial(
    pl.kernel, mesh=_mesh,
    compiler_params=pltpu.CompilerParams(use_tc_tiling_on_sc=False),
    out_type=jax.ShapeDtypeStruct((NC, NPAD, H), jnp.float32),
    scratch_types=[
        pltpu.VMEM((NCH, CH), jnp.int32),
        pltpu.VMEM((CH,), jnp.float32),
        pltpu.VMEM((ROWS_PS,), jnp.float32),
        pltpu.VMEM((ROWS_PS, H), jnp.float32),
        pltpu.VMEM_SHARED((NPAD,), jnp.float32),
        pltpu.SemaphoreType.DMA,
    ],
)
def _sc_degree(dst_hbm, deg_hbm, didx, ones, stage, stage2, dtab, sem):
    cid = lax.axis_index("c")
    sid = lax.axis_index("s")
    wid = sid * NC + cid
    for j in range(CH // 16):
        ones[pl.ds(j * 16, 16)] = jnp.full((16,), 1.0, jnp.float32)
    for j in range(ROWS_PS // 16):
        stage[pl.ds(j * 16, 16)] = jnp.zeros((16,), jnp.float32)
    pltpu.sync_copy(stage, dtab.at[pl.ds(sid * ROWS_PS, ROWS_PS)])
    pltpu.sync_copy(dst_hbm.at[wid], didx)
    plsc.subcore_barrier()

    def chunk(g, carry):
        base = g * 4
        cps = [
            pltpu.async_copy(ones, dtab.at[didx.at[base + k]], sem, add=True)
            for k in range(4)
        ]
        for cp in cps:
            cp.wait()
        return carry

    lax.fori_loop(0, NCH // 4, chunk, 0)
    plsc.subcore_barrier()
    pltpu.sync_copy(dtab.at[pl.ds(sid * ROWS_PS, ROWS_PS)], stage)

    def expand(j, carry):
        v = stage[pl.ds(j * 16, 16)]
        for l in range(16):
            stage2[j * 16 + l, :] = jnp.full((16,), v[l], jnp.float32)
        return carry

    lax.fori_loop(0, ROWS_PS // 16, expand, 0)
    pltpu.sync_copy(stage2, deg_hbm.at[cid, pl.ds(sid * ROWS_PS, ROWS_PS)])


@functools.partial(
    pl.kernel, mesh=_mesh,
    compiler_params=pltpu.CompilerParams(use_tc_tiling_on_sc=False),
    out_type=jax.ShapeDtypeStruct((NC, NPAD, H), jnp.float32),
    scratch_types=[
        pltpu.VMEM((NCH, CH), jnp.int32),
        pltpu.VMEM((NCH, CH), jnp.int32),
        [pltpu.VMEM((CH, H), jnp.float32)] * NBUF,
        pltpu.VMEM((ROWS_PS, H), jnp.float32),
        pltpu.VMEM_SHARED((NPAD, H), jnp.float32),
        pltpu.VMEM_SHARED((NPAD, H), jnp.float32),
        pltpu.SemaphoreType.DMA,
        pltpu.SemaphoreType.DMA,
    ],
)
def _sc_propagate(u_hbm, src_hbm, dst_hbm, s_hbm, sidx, didx, rows, stage,
                  stab, utab, gsem, ssem):
    cid = lax.axis_index("c")
    sid = lax.axis_index("s")
    wid = sid * NC + cid

    cp_s = pltpu.async_copy(src_hbm.at[wid], sidx, gsem)
    cp_d = pltpu.async_copy(dst_hbm.at[wid], didx, gsem)
    cp_u = pltpu.async_copy(u_hbm.at[pl.ds(sid * ROWS_PS, ROWS_PS)], stage,
                            ssem)

    def zrow(i, carry):
        rows[0][i % CH, :] = jnp.zeros((16,), jnp.float32)
        return carry

    lax.fori_loop(0, CH, zrow, 0)
    cp_u.wait()
    pltpu.sync_copy(stage, utab.at[pl.ds(sid * ROWS_PS, ROWS_PS)])
    for q in range(ROWS_PS // CH):
        pltpu.sync_copy(rows[0], stab.at[pl.ds(sid * ROWS_PS + q * CH, CH)])
    cp_s.wait()
    cp_d.wait()
    plsc.subcore_barrier()

    def group(g, carry):
        base = g * NBUF
        gs = [
            pltpu.async_copy(utab.at[sidx.at[base + k]], rows[k], gsem)
            for k in range(NBUF)
        ]
        scs = []
        for k in range(NBUF):
            gs[k].wait()
            scs.append(pltpu.async_copy(rows[k], stab.at[didx.at[base + k]],
                                        ssem, add=True))
        for cp in scs:
            cp.wait()
        return carry

    lax.fori_loop(0, NCH // NBUF, group, 0)
    plsc.subcore_barrier()
    pltpu.sync_copy(stab.at[pl.ds(sid * ROWS_PS, ROWS_PS)], stage)
    pltpu.sync_copy(stage, s_hbm.at[cid, pl.ds(sid * ROWS_PS, ROWS_PS)])


def _tc_first_body(xr_ref, w1s_ref, deg_ref, u_ref, de_ref):
    dinvexp = lax.rsqrt(deg_ref[0] + deg_ref[1] + 1.0)
    hp = jnp.dot(xr_ref[...], w1s_ref[...],
                 preferred_element_type=jnp.float32)
    u_ref[0:PRN, :] = hp * dinvexp[0:PRN, :]
    u_ref[PRN:, :] = jnp.zeros((PR - PRN, 128), jnp.float32)
    de_ref[...] = dinvexp


_tc_first = pl.pallas_call(
    _tc_first_body,
    out_shape=(jax.ShapeDtypeStruct((PR, 128), jnp.float32),
               jax.ShapeDtypeStruct((PR, 128), jnp.float32)),
)


def _tc_mid_body(s_ref, u_ref, de_ref, bt_ref, wbd_ref, un_ref):
    de = de_ref[...]
    ss = s_ref[0] + s_ref[1] + u_ref[...]
    r = jnp.maximum(de * ss + bt_ref[...], 0.0)
    un = de * jnp.dot(r, wbd_ref[...], preferred_element_type=jnp.float32)
    un_ref[0:PRN, :] = un[0:PRN, :]
    un_ref[PRN:, :] = jnp.zeros((PR - PRN, 128), jnp.float32)


_tc_mid = pl.pallas_call(
    _tc_mid_body,
    out_shape=jax.ShapeDtypeStruct((PR, 128), jnp.float32),
)


def _tc_final_body(s_ref, u_ref, de_ref, bt_ref, fold_ref, wf_ref, bf_ref,
                   out_ref):
    ss = s_ref[0] + s_ref[1] + u_ref[...]
    h4 = jnp.maximum(de_ref[...] * ss + bt_ref[...], 0.0)
    colsum = (jnp.sum(h4, axis=0, keepdims=True)
              - float(PR - PRN) * jnp.maximum(bt_ref[...], 0.0))
    pooled = jnp.dot(colsum, fold_ref[...],
                     preferred_element_type=jnp.float32) * (1.0 / N)
    logits = jnp.dot(pooled, wf_ref[...],
                     preferred_element_type=jnp.float32) + bf_ref[...]
    m = jnp.max(logits, axis=1, keepdims=True)
    ex = jnp.exp(logits - m)
    out_ref[...] = (logits - m) - jnp.log(jnp.sum(ex, axis=1, keepdims=True))


_tc_final = pl.pallas_call(
    _tc_final_body,
    out_shape=jax.ShapeDtypeStruct((1, 16), jnp.float32),
)


def kernel(x, edge_index, W1, b1, W2, b2, W3, b3, W4, b4, Wf, bf):
    ept = EPAD // TILES - E // TILES
    srcr = edge_index[0].astype(jnp.int32).reshape(TILES, E // TILES)
    dstr = edge_index[1].astype(jnp.int32).reshape(TILES, E // TILES)
    dumd = jnp.tile(jnp.arange(N, N + ept, dtype=jnp.int32)[None, :],
                    (TILES, 1))
    dums = dumd
    dst3 = jnp.concatenate([dstr, dumd], axis=1).reshape(TILES, NCH, CH)
    degexp = _sc_degree(dst3).reshape(NC, PR, 128)
    src3 = jnp.concatenate([srcr, dums], axis=1).reshape(TILES, NCH, CH)

    eye8 = jnp.eye(8, dtype=jnp.float32)
    w1s = jnp.kron(eye8, W1)
    fold = jnp.tile(jnp.eye(16, dtype=jnp.float32), (8, 1))

    u, dexp = _tc_first(x.reshape(PRN, 8 * D), w1s, degexp)

    for b, Wn in ((b1, W2), (b2, W3), (b3, W4)):
        s = _sc_propagate(u.reshape(NPAD, H), src3, dst3)
        u = _tc_mid(s.reshape(NC, PR, 128), u, dexp,
                    jnp.tile(b.reshape(1, H), (1, 8)), jnp.kron(eye8, Wn))
    s = _sc_propagate(u.reshape(NPAD, H), src3, dst3)

    return _tc_final(s.reshape(NC, PR, 128), u, dexp,
                     jnp.tile(b4.reshape(1, H), (1, 8)), fold, Wf,
                     bf.reshape(1, 16))
